# baseline (speedup 1.0000x reference)
import functools

import jax
import jax.numpy as jnp
from jax import lax
from jax.experimental import pallas as pl
from jax.experimental.pallas import tpu as pltpu
from jax.experimental.pallas import tpu_sc as plsc

NUM_CB = 4
TAB_ROWS = 1030
BT = 8
BATCH = 64
TIME = 2048
OUT_F = NUM_CB * BT
TC = 512
NC = 2
NS = 16
NW = NC * NS


def _body(idx_hbm, tab_hbm, out_hbm, idx_v, list_v, rows_v, sem):
  core = lax.axis_index("c")
  sub = lax.axis_index("s")
  wid = sub * NC + core

  iota = lax.iota(jnp.int32, 16)
  lane_c = jnp.bitwise_and(iota, 3)
  lane_t = jnp.right_shift(iota, 2)
  lane_off = lane_c * TAB_ROWS

  for bi in range(BATCH // NW):
    b = wid * (BATCH // NW) + bi
    for ck in range(TIME // TC):
      ts = ck * TC
      pltpu.sync_copy(idx_hbm.at[b, :, pl.ds(ts, TC)], idx_v)

      @pl.loop(0, TC // 4, unroll=8)
      def _(g):
        iv = plsc.load_gather(idx_v, [lane_c, lane_t + g * 4])
        list_v[pl.ds(g * 16, 16)] = iv + lane_off

      pltpu.async_copy(tab_hbm.at[list_v], rows_v, sem).wait()
      pltpu.sync_copy(rows_v, out_hbm.at[b, pl.ds(ts * NUM_CB, TC * NUM_CB), :])


@jax.jit
def _run(index_sequence, tab2d):
  mesh = plsc.VectorSubcoreMesh(core_axis_name="c", subcore_axis_name="s")
  fn = pl.kernel(
      _body,
      out_type=jax.ShapeDtypeStruct((BATCH, TIME * NUM_CB, BT), jnp.float32),
      mesh=mesh,
      scratch_types=[
          pltpu.VMEM((NUM_CB, TC), jnp.int32),
          pltpu.VMEM((TC * NUM_CB,), jnp.int32),
          pltpu.VMEM((TC * NUM_CB, BT), jnp.float32),
          pltpu.SemaphoreType.DMA,
      ],
      compiler_params=pltpu.CompilerParams(
          needs_layout_passes=False, use_tc_tiling_on_sc=False),
  )
  return fn(index_sequence, tab2d)


def kernel(index_sequence, speaker_embedding, tables, is_inference):
  del speaker_embedding, is_inference
  tab2d = tables.reshape(NUM_CB * TAB_ROWS, BT)
  out = _run(index_sequence, tab2d)
  return out.reshape(BATCH, TIME, OUT_F)


# speedup vs baseline: 1.9654x; 1.9654x over previous
import functools

import jax
import jax.numpy as jnp
from jax import lax
from jax.experimental import pallas as pl
from jax.experimental.pallas import tpu as pltpu
from jax.experimental.pallas import tpu_sc as plsc

NUM_CB = 4
TAB_ROWS = 1030
BT = 8
BATCH = 64
TIME = 2048
OUT_F = NUM_CB * BT
TC = 512
NC = 2
NS = 16
NW = NC * NS
N_CHUNK = (BATCH // NW) * (TIME // TC)


def _body(idx_hbm, tab_hbm, out_hbm, table_v, idx_v0, idx_v1, out_v0, out_v1,
          idx_sem, out_sem):
  idx_bufs = (idx_v0, idx_v1)
  out_bufs = (out_v0, out_v1)
  core = lax.axis_index("c")
  sub = lax.axis_index("s")
  wid = sub * NC + core
  b0 = wid * (BATCH // NW)

  pltpu.sync_copy(tab_hbm, table_v)

  iota32 = lax.iota(jnp.int32, 16) * OUT_F

  def chunk_coords(k):
    b = b0 + k // (TIME // TC)
    ts = (k % (TIME // TC)) * TC
    return b, ts

  def start_idx(k):
    b, ts = chunk_coords(k)
    return pltpu.async_copy(
        idx_hbm.at[b, :, pl.ds(ts, TC)], idx_bufs[k % 2], idx_sem)

  def start_out(k):
    b, ts = chunk_coords(k)
    return pltpu.async_copy(
        out_bufs[k % 2], out_hbm.at[b, pl.ds(ts * OUT_F, TC * OUT_F)], out_sem)

  idx_dma = [start_idx(0)]
  out_dma = []
  for k in range(N_CHUNK):
    if k + 1 < N_CHUNK:
      idx_dma.append(start_idx(k + 1))
    idx_dma[k].wait()
    if k >= 2:
      out_dma[k - 2].wait()
    ib = idx_bufs[k % 2]
    ob = out_bufs[k % 2]

    @plsc.parallel_loop(0, TC // 16, unroll=4)
    def _(t16):
      t0 = pl.multiple_of(t16 * 16, 16)
      for c in range(NUM_CB):
        iv = ib[c, pl.ds(t0, 16)]
        base = iv * BT + c * (TAB_ROWS * BT)
        for d in range(BT):
          val = plsc.load_gather(table_v, [base + d])
          plsc.store_scatter(ob, [iota32 + (t0 * OUT_F + c * BT + d)], val)

    out_dma.append(start_out(k))
  out_dma[N_CHUNK - 2].wait()
  out_dma[N_CHUNK - 1].wait()


@jax.jit
def _run(index_sequence, tab_flat):
  mesh = plsc.VectorSubcoreMesh(core_axis_name="c", subcore_axis_name="s")
  fn = pl.kernel(
      _body,
      out_type=jax.ShapeDtypeStruct((BATCH, TIME * OUT_F), jnp.float32),
      mesh=mesh,
      scratch_types=[
          pltpu.VMEM((NUM_CB * TAB_ROWS * BT,), jnp.float32),
          pltpu.VMEM((NUM_CB, TC), jnp.int32),
          pltpu.VMEM((NUM_CB, TC), jnp.int32),
          pltpu.VMEM((TC * OUT_F,), jnp.float32),
          pltpu.VMEM((TC * OUT_F,), jnp.float32),
          pltpu.SemaphoreType.DMA,
          pltpu.SemaphoreType.DMA,
      ],
      compiler_params=pltpu.CompilerParams(needs_layout_passes=False),
  )
  return fn(index_sequence, tab_flat)


def kernel(index_sequence, speaker_embedding, tables, is_inference):
  del speaker_embedding, is_inference
  tab_flat = tables.reshape(-1)
  out = _run(index_sequence, tab_flat)
  return out.reshape(BATCH, TIME, OUT_F)


# speedup vs baseline: 2.1855x; 1.1120x over previous
import functools

import jax
import jax.numpy as jnp
from jax import lax
from jax.experimental import pallas as pl
from jax.experimental.pallas import tpu as pltpu
from jax.experimental.pallas import tpu_sc as plsc

NUM_CB = 4
TAB_ROWS = 1030
BT = 8
BATCH = 64
TIME = 2048
OUT_F = NUM_CB * BT
TC = 512
NC = 2
NS = 16
NW = NC * NS
N_CHUNK = (BATCH // NW) * (TIME // TC)


def _body(idx_hbm, tab_hbm, out_hbm, table_v, idx_v0, idx_v1, out_v0, out_v1,
          idx_sem, out_sem):
  idx_bufs = (idx_v0, idx_v1)
  out_bufs = (out_v0, out_v1)
  core = lax.axis_index("c")
  sub = lax.axis_index("s")
  wid = sub * NC + core
  b0 = wid * (BATCH // NW)

  pltpu.sync_copy(tab_hbm, table_v)

  iota = lax.iota(jnp.int32, 16)
  dvec = jnp.bitwise_and(iota, 7)
  m8 = iota < 8
  cd01 = jnp.where(m8, 0, TAB_ROWS * BT) + dvec
  cd23 = jnp.where(m8, 2 * TAB_ROWS * BT, 3 * TAB_ROWS * BT) + dvec
  rw01 = jnp.where(m8, 0, 1)
  rw23 = jnp.where(m8, 2, 3)

  def chunk_coords(k):
    b = b0 + k // (TIME // TC)
    ts = (k % (TIME // TC)) * TC
    return b, ts

  def start_idx(k):
    b, ts = chunk_coords(k)
    return pltpu.async_copy(
        idx_hbm.at[b, :, pl.ds(ts, TC)], idx_bufs[k % 2], idx_sem)

  def start_out(k):
    b, ts = chunk_coords(k)
    return pltpu.async_copy(
        out_bufs[k % 2], out_hbm.at[b, pl.ds(ts * OUT_F, TC * OUT_F)], out_sem)

  idx_dma = [start_idx(0)]
  out_dma = []
  for k in range(N_CHUNK):
    if k + 1 < N_CHUNK:
      idx_dma.append(start_idx(k + 1))
    idx_dma[k].wait()
    ib = idx_bufs[k % 2]
    if k >= 2:
      out_dma[k - 2].wait()
    ob = out_bufs[k % 2]

    @plsc.parallel_loop(0, TC, unroll=8)
    def _(t):
      t32 = pl.multiple_of(t * OUT_F, 16)
      tvec = jnp.full((16,), t, jnp.int32)
      iv01 = plsc.load_gather(ib, [rw01, tvec])
      iv23 = plsc.load_gather(ib, [rw23, tvec])
      g0 = plsc.load_gather(table_v, [iv01 * BT + cd01])
      g1 = plsc.load_gather(table_v, [iv23 * BT + cd23])
      ob[pl.ds(t32, 16)] = g0
      ob[pl.ds(t32 + 16, 16)] = g1

    out_dma.append(start_out(k))
  out_dma[N_CHUNK - 2].wait()
  out_dma[N_CHUNK - 1].wait()


@jax.jit
def _run(index_sequence, tab_flat):
  mesh = plsc.VectorSubcoreMesh(core_axis_name="c", subcore_axis_name="s")
  fn = pl.kernel(
      _body,
      out_type=jax.ShapeDtypeStruct((BATCH, TIME * OUT_F), jnp.float32),
      mesh=mesh,
      scratch_types=[
          pltpu.VMEM((NUM_CB * TAB_ROWS * BT,), jnp.float32),
          pltpu.VMEM((NUM_CB, TC), jnp.int32),
          pltpu.VMEM((NUM_CB, TC), jnp.int32),
          pltpu.VMEM((TC * OUT_F,), jnp.float32),
          pltpu.VMEM((TC * OUT_F,), jnp.float32),
          pltpu.SemaphoreType.DMA,
          pltpu.SemaphoreType.DMA,
      ],
      compiler_params=pltpu.CompilerParams(needs_layout_passes=False),
  )
  return fn(index_sequence, tab_flat)


def kernel(index_sequence, speaker_embedding, tables, is_inference):
  del speaker_embedding, is_inference
  tab_flat = tables.reshape(-1)
  out = _run(index_sequence, tab_flat)
  return out.reshape(BATCH, TIME, OUT_F)


# speedup vs baseline: 5.1150x; 2.3405x over previous
import functools

import jax
import jax.numpy as jnp
from jax import lax
from jax.experimental import pallas as pl
from jax.experimental.pallas import tpu as pltpu
from jax.experimental.pallas import tpu_sc as plsc

NUM_CB = 4
TAB_ROWS = 1030
BT = 8
BATCH = 64
TIME = 2048
OUT_F = NUM_CB * BT
NC = 2
NS = 16
NW = NC * NS
N_UNIT = (BATCH // NW) * NUM_CB
TT = TIME // 128


def _body(idx_hbm, tab_hbm, out_hbm, table_v, idx_v0, idx_v1, out_v0, out_v1,
          idx_sem, out_sem):
  idx_bufs = (idx_v0, idx_v1)
  out_bufs = (out_v0, out_v1)
  core = lax.axis_index("c")
  sub = lax.axis_index("s")
  wid = sub * NC + core
  b0 = wid * (BATCH // NW)

  pltpu.sync_copy(tab_hbm, table_v)

  def unit_coords(k):
    return b0 + k // NUM_CB, k % NUM_CB

  def start_idx(k):
    b, c = unit_coords(k)
    return pltpu.async_copy(idx_hbm.at[b, c, :], idx_bufs[k % 2], idx_sem)

  def start_out(k):
    b, c = unit_coords(k)
    return pltpu.async_copy(out_bufs[k % 2], out_hbm.at[b, c], out_sem)

  idx_dma = [start_idx(0)]
  out_dma = []
  for k in range(N_UNIT):
    if k + 1 < N_UNIT:
      idx_dma.append(start_idx(k + 1))
    idx_dma[k].wait()
    ib = idx_bufs[k % 2]
    if k >= 2:
      out_dma[k - 2].wait()
    ob = out_bufs[k % 2]
    _, c = unit_coords(k)

    @plsc.parallel_loop(0, TIME // 16, unroll=4)
    def _(g):
      t0 = pl.multiple_of(g * 16, 16)
      base = pl.multiple_of((g // 8) * 1024 + (g % 8) * 16, 16)
      iv = ib[pl.ds(t0, 16)] + c * (TAB_ROWS * BT)
      for d in range(BT):
        val = plsc.load_gather(table_v, [iv + d * TAB_ROWS])
        ob[pl.ds(base + d * 128, 16)] = val

    out_dma.append(start_out(k))
  out_dma[N_UNIT - 2].wait()
  out_dma[N_UNIT - 1].wait()


@jax.jit
def _run(index_sequence, tab_fmajor):
  mesh = plsc.VectorSubcoreMesh(core_axis_name="c", subcore_axis_name="s")
  fn = pl.kernel(
      _body,
      out_type=jax.ShapeDtypeStruct((BATCH, NUM_CB, TT * BT * 128), jnp.float32),
      mesh=mesh,
      scratch_types=[
          pltpu.VMEM((NUM_CB * TAB_ROWS * BT,), jnp.float32),
          pltpu.VMEM((TIME,), jnp.int32),
          pltpu.VMEM((TIME,), jnp.int32),
          pltpu.VMEM((TT * BT * 128,), jnp.float32),
          pltpu.VMEM((TT * BT * 128,), jnp.float32),
          pltpu.SemaphoreType.DMA,
          pltpu.SemaphoreType.DMA,
      ],
      compiler_params=pltpu.CompilerParams(needs_layout_passes=False),
  )
  return fn(index_sequence, tab_fmajor)


def kernel(index_sequence, speaker_embedding, tables, is_inference):
  del speaker_embedding, is_inference
  tab_fmajor = jnp.transpose(tables, (0, 2, 1)).reshape(-1)
  out = _run(index_sequence, tab_fmajor)
  out = out.reshape(BATCH, NUM_CB, TT, BT, 128)
  return jnp.transpose(out, (0, 2, 4, 1, 3)).reshape(BATCH, TIME, OUT_F)


# speedup vs baseline: 8.3555x; 1.6335x over previous
import functools

import jax
import jax.numpy as jnp
from jax import lax
from jax.experimental import pallas as pl
from jax.experimental.pallas import tpu as pltpu
from jax.experimental.pallas import tpu_sc as plsc

NUM_CB = 4
TAB_ROWS = 1030
BT = 8
BATCH = 64
TIME = 2048
OUT_F = NUM_CB * BT
NC = 2
NS = 16
NW = NC * NS
N_UNIT = (BATCH // NW) * NUM_CB
TT = TIME // 128


def _body(idx_hbm, tab_hbm, out_hbm, table_v, idx_v0, idx_v1, out_v0, out_v1,
          idx_sem, out_sem):
  idx_bufs = (idx_v0, idx_v1)
  out_bufs = (out_v0, out_v1)
  core = lax.axis_index("c")
  sub = lax.axis_index("s")
  wid = sub * NC + core
  b0 = wid * (BATCH // NW)

  pltpu.sync_copy(tab_hbm, table_v)

  def unit_coords(k):
    return b0 + k // NUM_CB, k % NUM_CB

  def start_idx(k):
    b, c = unit_coords(k)
    return pltpu.async_copy(idx_hbm.at[b, c, :], idx_bufs[k % 2], idx_sem)

  def start_out(k):
    b, c = unit_coords(k)
    return pltpu.async_copy(out_bufs[k % 2], out_hbm.at[b, c], out_sem)

  idx_dma = [start_idx(0)]
  out_dma = []
  for k in range(N_UNIT):
    if k + 1 < N_UNIT:
      idx_dma.append(start_idx(k + 1))
    idx_dma[k].wait()
    ib = idx_bufs[k % 2]
    if k >= 2:
      out_dma[k - 2].wait()
    ob = out_bufs[k % 2]
    _, c = unit_coords(k)

    @plsc.parallel_loop(0, TIME // 16, unroll=4)
    def _(g):
      t0 = pl.multiple_of(g * 16, 16)
      tt = g // 8
      tl0 = pl.multiple_of((g % 8) * 16, 16)
      iv = ib[pl.ds(t0, 16)] + c * (TAB_ROWS * BT)
      for d in range(BT):
        val = plsc.load_gather(table_v, [iv + d * TAB_ROWS])
        ob[tt, d, pl.ds(tl0, 16)] = val

    out_dma.append(start_out(k))
  out_dma[N_UNIT - 2].wait()
  out_dma[N_UNIT - 1].wait()


@jax.jit
def _run(index_sequence, tab_fmajor):
  mesh = plsc.VectorSubcoreMesh(core_axis_name="c", subcore_axis_name="s")
  fn = pl.kernel(
      _body,
      out_type=jax.ShapeDtypeStruct((BATCH, NUM_CB, TT, BT, 128), jnp.float32),
      mesh=mesh,
      scratch_types=[
          pltpu.VMEM((NUM_CB * TAB_ROWS * BT,), jnp.float32),
          pltpu.VMEM((TIME,), jnp.int32),
          pltpu.VMEM((TIME,), jnp.int32),
          pltpu.VMEM((TT, BT, 128), jnp.float32),
          pltpu.VMEM((TT, BT, 128), jnp.float32),
          pltpu.SemaphoreType.DMA,
          pltpu.SemaphoreType.DMA,
      ],
      compiler_params=pltpu.CompilerParams(needs_layout_passes=False),
  )
  return fn(index_sequence, tab_fmajor)


def kernel(index_sequence, speaker_embedding, tables, is_inference):
  del speaker_embedding, is_inference
  tab_fmajor = jnp.transpose(tables, (0, 2, 1)).reshape(-1)
  out = _run(index_sequence, tab_fmajor)
  return jnp.transpose(out, (0, 2, 4, 1, 3)).reshape(BATCH, TIME, OUT_F)
